# baseline (speedup 1.0000x reference)
import functools

import jax
import jax.numpy as jnp
from jax.experimental import pallas as pl
from jax.experimental.pallas import tpu as pltpu

N = 100000
C = 64
KEY = 32
H = 1
VAL = 1 + C
QKV = 2 * KEY + VAL
TOPK = 32


def _ln(x, g, b, eps=1e-5):
    mu = jnp.mean(x, axis=-1, keepdims=True)
    var = jnp.var(x, axis=-1, keepdims=True)
    return (x - mu) / jnp.sqrt(var + eps) * g + b


def _attn_body(deltas_ref, gath_ref, Wq_ref, bq_ref, g1_ref, b1_ref,
               gm_ref, bm_ref, W1_ref, b1m_ref, W2_ref, b2m_ref,
               sW1_ref, sb1_ref, sW2_ref, sb2_ref, out_ref):
    bblk = deltas_ref.shape[0]
    deltas = deltas_ref[...]
    gath = gath_ref[...]
    retrieved = jnp.concatenate([deltas[..., None], gath], axis=-1)

    flat = retrieved.reshape(bblk * TOPK, VAL)
    qkv = jnp.dot(flat, Wq_ref[...], preferred_element_type=jnp.float32)
    qkv = qkv + bq_ref[...]
    qkv = _ln(qkv, g1_ref[...], b1_ref[...])
    qkv = qkv.reshape(bblk, TOPK, QKV)

    q = qkv[..., :KEY] * (KEY ** -0.5)
    kk = qkv[..., KEY:2 * KEY]
    v = qkv[..., 2 * KEY:]

    logits = jax.lax.dot_general(
        q, kk, (((2,), (2,)), ((0,), (0,))),
        preferred_element_type=jnp.float32)
    w = jax.nn.softmax(logits, axis=-1)
    attended = jax.lax.dot_general(
        w, v, (((2,), (1,)), ((0,), (0,))),
        preferred_element_type=jnp.float32)

    mem = _ln((retrieved + attended).reshape(bblk * TOPK, VAL),
              gm_ref[...], bm_ref[...])
    h = jnp.dot(jax.nn.relu(jnp.dot(mem, W1_ref[...],
                                    preferred_element_type=jnp.float32)
                            + b1m_ref[...]),
                W2_ref[...], preferred_element_type=jnp.float32) + b2m_ref[...]
    mem = _ln(h + mem, gm_ref[...], bm_ref[...])
    h = jnp.dot(jax.nn.relu(jnp.dot(mem, W1_ref[...],
                                    preferred_element_type=jnp.float32)
                            + b1m_ref[...]),
                W2_ref[...], preferred_element_type=jnp.float32) + b2m_ref[...]
    mem = _ln(h + mem, gm_ref[...], bm_ref[...])
    cp = jnp.dot(jax.nn.relu(jnp.dot(mem, sW1_ref[...],
                                     preferred_element_type=jnp.float32)
                             + sb1_ref[...]),
                 sW2_ref[...], preferred_element_type=jnp.float32) + sb2_ref[...]
    out_ref[...] = cp.reshape(bblk, TOPK, C)


def _attend(deltas, gathered, W_qkv, b_qkv, ln1_g, ln1_b, lnm_g, lnm_b,
            mlp_W1, mlp_b1, mlp_W2, mlp_b2, skip_W1, skip_b1, skip_W2, skip_b2):
    B = deltas.shape[0]
    BBLK = 256
    grid = (B // BBLK,)
    full = lambda shape: pl.BlockSpec(shape, lambda i: (0,) * len(shape))
    return pl.pallas_call(
        _attn_body,
        grid=grid,
        in_specs=[
            pl.BlockSpec((BBLK, TOPK), lambda i: (i, 0)),
            pl.BlockSpec((BBLK, TOPK, C), lambda i: (i, 0, 0)),
            full((VAL, QKV)), full((QKV,)), full((QKV,)), full((QKV,)),
            full((VAL,)), full((VAL,)),
            full((VAL, VAL)), full((VAL,)), full((VAL, VAL)), full((VAL,)),
            full((VAL, VAL)), full((VAL,)), full((VAL, C)), full((C,)),
        ],
        out_specs=pl.BlockSpec((BBLK, TOPK, C), lambda i: (i, 0, 0)),
        out_shape=jax.ShapeDtypeStruct((B, TOPK, C), jnp.float32),
    )(deltas, gathered, W_qkv, b_qkv, ln1_g, ln1_b, lnm_g, lnm_b,
      mlp_W1, mlp_b1, mlp_W2, mlp_b2, skip_W1, skip_b1, skip_W2, skip_b2)


def kernel(c, mem_c, W_qkv, b_qkv, ln1_g, ln1_b, lnm_g, lnm_b,
           mlp_W1, mlp_b1, mlp_W2, mlp_b2,
           skip_W1, skip_b1, skip_W2, skip_b2, k):
    tau = c @ mem_c.T
    deltas, indices = jax.lax.top_k(tau, TOPK)
    gathered = jnp.take(mem_c, indices, axis=0)
    return _attend(deltas, gathered, W_qkv, b_qkv, ln1_g, ln1_b, lnm_g, lnm_b,
                   mlp_W1, mlp_b1, mlp_W2, mlp_b2,
                   skip_W1, skip_b1, skip_W2, skip_b2)


# speedup vs baseline: 11.9433x; 11.9433x over previous
import functools

import jax
import jax.numpy as jnp
from jax import lax
from jax.experimental import pallas as pl
from jax.experimental.pallas import tpu as pltpu
from jax.experimental.pallas import tpu_sc as plsc

B = 1024
N = 100000
C = 64
KEY = 32
VAL = 1 + C
QKV = 2 * KEY + VAL
TOPK = 32
GROUP = 128
CHUNK = 2048
NPAD = 100352
NCHUNKS = NPAD // CHUNK
NGROUPS = NPAD // GROUP
GPC = CHUNK // GROUP
NEG = float("-inf")

SC_CORES = 2
SC_SUBCORES = 16
SC_NW = SC_CORES * SC_SUBCORES


def _sim_body(c_ref, memb_ref, tau_ref, g_ref):
    j = pl.program_id(0)
    scores = lax.dot_general(c_ref[...], memb_ref[...],
                             (((1,), (1,)), ((), ())),
                             preferred_element_type=jnp.float32)
    col = j * CHUNK + lax.broadcasted_iota(jnp.int32, (B, CHUNK), 1)
    scores = jnp.where(col < N, scores, NEG)
    tau_ref[...] = scores.reshape(B, GPC, GROUP)
    g_ref[0] = jnp.max(scores.reshape(B, GPC, GROUP), axis=-1)


def _similarity(c, mem_pad):
    return pl.pallas_call(
        _sim_body,
        grid=(NCHUNKS,),
        in_specs=[
            pl.BlockSpec((B, C), lambda j: (0, 0)),
            pl.BlockSpec((CHUNK, C), lambda j: (j, 0)),
        ],
        out_specs=[
            pl.BlockSpec((B, GPC, GROUP), lambda j: (0, j, 0)),
            pl.BlockSpec((1, B, GPC), lambda j: (j, 0, 0)),
        ],
        out_shape=[
            jax.ShapeDtypeStruct((B, NGROUPS, GROUP), jnp.float32),
            jax.ShapeDtypeStruct((NCHUNKS, B, GPC), jnp.float32),
        ],
    )(c, mem_pad)


def _topk_groups_body(g_ref, gid_ref):
    x = g_ref[...]
    bblk = x.shape[0]
    gid = lax.broadcasted_iota(jnp.int32, (bblk, NGROUPS), 1)
    sels = []
    for _ in range(TOPK):
        m = jnp.max(x, axis=1, keepdims=True)
        sel = jnp.min(jnp.where(x == m, gid, NGROUPS), axis=1, keepdims=True)
        sels.append(sel)
        x = jnp.where(gid == sel, NEG, x)
    gid_ref[...] = jnp.concatenate(sels, axis=1)


def _topk_groups(G):
    BBLK = 256
    return pl.pallas_call(
        _topk_groups_body,
        grid=(B // BBLK,),
        in_specs=[pl.BlockSpec((BBLK, NGROUPS), lambda i: (i, 0))],
        out_specs=pl.BlockSpec((BBLK, TOPK), lambda i: (i, 0)),
        out_shape=jax.ShapeDtypeStruct((B, TOPK), jnp.int32),
    )(G)


def _topk_cands_body(cand_ref, wgid_ref, d_ref, i_ref):
    x = cand_ref[...]
    bblk = x.shape[0]
    lane = lax.broadcasted_iota(jnp.int32, (bblk, TOPK, GROUP), 2)
    gcol = (wgid_ref[...][:, :, None] * GROUP + lane).reshape(bblk, TOPK * GROUP)
    ds, gs = [], []
    for _ in range(TOPK):
        m = jnp.max(x, axis=1, keepdims=True)
        sel = jnp.min(jnp.where(x == m, gcol, 2 ** 30), axis=1, keepdims=True)
        ds.append(m)
        gs.append(sel)
        x = jnp.where(gcol == sel, NEG, x)
    d_ref[...] = jnp.concatenate(ds, axis=1)
    i_ref[...] = jnp.concatenate(gs, axis=1)


def _topk_cands(cand, wgid):
    BBLK = 256
    return pl.pallas_call(
        _topk_cands_body,
        grid=(B // BBLK,),
        in_specs=[
            pl.BlockSpec((BBLK, TOPK * GROUP), lambda i: (i, 0)),
            pl.BlockSpec((BBLK, TOPK), lambda i: (i, 0)),
        ],
        out_specs=[
            pl.BlockSpec((BBLK, TOPK), lambda i: (i, 0)),
            pl.BlockSpec((BBLK, TOPK), lambda i: (i, 0)),
        ],
        out_shape=[
            jax.ShapeDtypeStruct((B, TOPK), jnp.float32),
            jax.ShapeDtypeStruct((B, TOPK), jnp.int32),
        ],
    )(cand, wgid)


def _sc_gather(table, idx):
    V, D = table.shape
    Btot = idx.shape[0]
    b_per_w = Btot // SC_NW
    nch = b_per_w // 128
    idx3 = idx.reshape(SC_NW, nch, 128)
    mesh = plsc.VectorSubcoreMesh(core_axis_name="c", subcore_axis_name="s")

    @functools.partial(
        pl.kernel,
        out_type=jax.ShapeDtypeStruct((Btot, D), jnp.float32),
        mesh=mesh,
        scratch_types=[
            pltpu.VMEM((nch, 128), jnp.int32),
            pltpu.VMEM((128, D), jnp.float32),
            pltpu.VMEM((128, D), jnp.float32),
            pltpu.SemaphoreType.DMA,
            pltpu.SemaphoreType.DMA,
        ],
    )
    def k(table_hbm, idx_hbm, out_hbm, idx_v, buf0, buf1, sem0, sem1):
        wid = lax.axis_index("s") * SC_CORES + lax.axis_index("c")
        base = wid * b_per_w
        pltpu.sync_copy(idx_hbm.at[wid], idx_v)
        bufs = (buf0, buf1)
        sems = (sem0, sem1)
        cps = [None, None]
        for t in range(nch):
            cps[t % 2] = pltpu.async_copy(table_hbm.at[idx_v.at[t]],
                                          bufs[t % 2], sems[t % 2])
            if t > 0:
                cps[(t - 1) % 2].wait()
                pltpu.sync_copy(bufs[(t - 1) % 2],
                                out_hbm.at[pl.ds(base + (t - 1) * 128, 128)])
        cps[(nch - 1) % 2].wait()
        pltpu.sync_copy(bufs[(nch - 1) % 2],
                        out_hbm.at[pl.ds(base + (nch - 1) * 128, 128)])

    return k(table, idx3)


def _ln(x, g, b, eps=1e-5):
    mu = jnp.mean(x, axis=-1, keepdims=True)
    var = jnp.var(x, axis=-1, keepdims=True)
    return (x - mu) / jnp.sqrt(var + eps) * g + b


def _attn_body(deltas_ref, gath_ref, Wq_ref, bq_ref, g1_ref, b1_ref,
               gm_ref, bm_ref, W1_ref, b1m_ref, W2_ref, b2m_ref,
               sW1_ref, sb1_ref, sW2_ref, sb2_ref, out_ref):
    bblk = deltas_ref.shape[0]
    deltas = deltas_ref[...]
    gath = gath_ref[..., :C]
    retrieved = jnp.concatenate([deltas[..., None], gath], axis=-1)

    flat = retrieved.reshape(bblk * TOPK, VAL)
    qkv = jnp.dot(flat, Wq_ref[...], preferred_element_type=jnp.float32)
    qkv = qkv + bq_ref[...]
    qkv = _ln(qkv, g1_ref[...], b1_ref[...])
    qkv = qkv.reshape(bblk, TOPK, QKV)

    q = qkv[..., :KEY] * (KEY ** -0.5)
    kk = qkv[..., KEY:2 * KEY]
    v = qkv[..., 2 * KEY:]

    logits = lax.dot_general(q, kk, (((2,), (2,)), ((0,), (0,))),
                             preferred_element_type=jnp.float32)
    w = jax.nn.softmax(logits, axis=-1)
    attended = lax.dot_general(w, v, (((2,), (1,)), ((0,), (0,))),
                               preferred_element_type=jnp.float32)

    mem = _ln((retrieved + attended).reshape(bblk * TOPK, VAL),
              gm_ref[...], bm_ref[...])
    h = jnp.dot(jax.nn.relu(jnp.dot(mem, W1_ref[...],
                                    preferred_element_type=jnp.float32)
                            + b1m_ref[...]),
                W2_ref[...], preferred_element_type=jnp.float32) + b2m_ref[...]
    mem = _ln(h + mem, gm_ref[...], bm_ref[...])
    h = jnp.dot(jax.nn.relu(jnp.dot(mem, W1_ref[...],
                                    preferred_element_type=jnp.float32)
                            + b1m_ref[...]),
                W2_ref[...], preferred_element_type=jnp.float32) + b2m_ref[...]
    mem = _ln(h + mem, gm_ref[...], bm_ref[...])
    cp = jnp.dot(jax.nn.relu(jnp.dot(mem, sW1_ref[...],
                                     preferred_element_type=jnp.float32)
                             + sb1_ref[...]),
                 sW2_ref[...], preferred_element_type=jnp.float32) + sb2_ref[...]
    out_ref[...] = cp.reshape(bblk, TOPK, C)


def _attend(deltas, gathered, W_qkv, b_qkv, ln1_g, ln1_b, lnm_g, lnm_b,
            mlp_W1, mlp_b1, mlp_W2, mlp_b2, skip_W1, skip_b1, skip_W2, skip_b2):
    BBLK = 256
    full = lambda shape: pl.BlockSpec(shape, lambda i: (0,) * len(shape))
    return pl.pallas_call(
        _attn_body,
        grid=(B // BBLK,),
        in_specs=[
            pl.BlockSpec((BBLK, TOPK), lambda i: (i, 0)),
            pl.BlockSpec((BBLK, TOPK, 2 * C), lambda i: (i, 0, 0)),
            full((VAL, QKV)), full((QKV,)), full((QKV,)), full((QKV,)),
            full((VAL,)), full((VAL,)),
            full((VAL, VAL)), full((VAL,)), full((VAL, VAL)), full((VAL,)),
            full((VAL, VAL)), full((VAL,)), full((VAL, C)), full((C,)),
        ],
        out_specs=pl.BlockSpec((BBLK, TOPK, C), lambda i: (i, 0, 0)),
        out_shape=jax.ShapeDtypeStruct((B, TOPK, C), jnp.float32),
    )(deltas, gathered, W_qkv, b_qkv, ln1_g, ln1_b, lnm_g, lnm_b,
      mlp_W1, mlp_b1, mlp_W2, mlp_b2, skip_W1, skip_b1, skip_W2, skip_b2)


def kernel(c, mem_c, W_qkv, b_qkv, ln1_g, ln1_b, lnm_g, lnm_b,
           mlp_W1, mlp_b1, mlp_W2, mlp_b2,
           skip_W1, skip_b1, skip_W2, skip_b2, k):
    mem_pad = jnp.pad(mem_c, ((0, NPAD - N), (0, 0)))
    tau, G3 = _similarity(c, mem_pad)
    G = G3.transpose(1, 0, 2).reshape(B, NGROUPS)
    wgid = _topk_groups(G)
    rowbase = jnp.arange(B, dtype=jnp.int32)[:, None] * NGROUPS
    cand = _sc_gather(tau.reshape(B * NGROUPS, GROUP),
                      (rowbase + wgid).reshape(-1))
    deltas, idx = _topk_cands(cand.reshape(B, TOPK * GROUP), wgid)
    mem_c128 = jnp.pad(mem_c, ((0, 0), (0, 2 * C - C)))
    gathered = _sc_gather(mem_c128, idx.reshape(-1)).reshape(B, TOPK, 2 * C)
    return _attend(deltas, gathered, W_qkv, b_qkv, ln1_g, ln1_b, lnm_g, lnm_b,
                   mlp_W1, mlp_b1, mlp_W2, mlp_b2,
                   skip_W1, skip_b1, skip_W2, skip_b2)
